# baseline (speedup 1.0000x reference)
import functools

import jax
import jax.numpy as jnp
from jax import lax
from jax.experimental import pallas as pl
from jax.experimental.pallas import tpu as pltpu
from jax.experimental.pallas import tpu_sc as plsc

N_NODES = 10000
N_EDGES = 160000
D = 256
H = D // 2
NS = 16
CHUNK = 80
EDGES_PER_TILE = N_EDGES // NS
CHUNKS_PER_TILE = EDGES_PER_TILE // CHUNK
NBUF = 4
NAUX = 8
N_PAD = 10240
ROWS_PER_TILE = N_PAD // NS
ROW_BLK = 1000
LANES = 16


def _weights_kl_body(mu1, ls1, e1, mu2, ls2, e2, mu3, ls3, e3, mu4, ls4, e4,
                     w1, w2, w3, w4, kl):
    tot = jnp.float32(0.0)
    for m, l, e, w in ((mu1, ls1, e1, w1), (mu2, ls2, e2, w2),
                       (mu3, ls3, e3, w3), (mu4, ls4, e4, w4)):
        lv = l[...]
        mv = m[...]
        w[...] = mv + e[...] * jnp.exp(lv)
        tot = tot + jnp.sum(0.5 * (jnp.exp(2.0 * lv) + mv * mv - 2.0 * lv - 1.0))
    kl[...] = jnp.reshape(tot, (1, 1))


def _weights_kl(mus_ls_eps):
    outs = [jax.ShapeDtypeStruct((D, D), jnp.float32) for _ in range(4)]
    outs.append(jax.ShapeDtypeStruct((1, 1), jnp.float32))
    return pl.pallas_call(
        _weights_kl_body,
        out_shape=tuple(outs),
    )(*mus_ls_eps)


def _halves_spec():
    return pl.BlockSpec((2, ROW_BLK, H), lambda i: (0, i, 0))


def _mm1_body(x_ref, w1_ref, w2_ref, o1, o2):
    xv = x_ref[...]
    s1 = jnp.dot(xv, w1_ref[...], preferred_element_type=jnp.float32)
    s2 = jnp.dot(xv, w2_ref[...], preferred_element_type=jnp.float32)
    o1[0] = s1[:, :H]
    o1[1] = s1[:, H:]
    o2[0] = s2[:, :H]
    o2[1] = s2[:, H:]


def _mm1(x, w1, w2):
    grid = (N_NODES // ROW_BLK,)
    return pl.pallas_call(
        _mm1_body,
        grid=grid,
        in_specs=[
            pl.BlockSpec((ROW_BLK, D), lambda i: (i, 0)),
            pl.BlockSpec((D, D), lambda i: (0, 0)),
            pl.BlockSpec((D, D), lambda i: (0, 0)),
        ],
        out_specs=[_halves_spec()] * 2,
        out_shape=tuple(jax.ShapeDtypeStruct((2, N_NODES, H), jnp.float32)
                        for _ in range(2)),
    )(x, w1, w2)


def _mm2_body(m_ref, iv_ref, w3_ref, w4_ref, o3, o4):
    w3 = w3_ref[...]
    w4 = w4_ref[...]
    s3 = (jnp.dot(m_ref[0], w3[:H, :], preferred_element_type=jnp.float32)
          + jnp.dot(m_ref[1], w3[H:, :], preferred_element_type=jnp.float32))
    va = jnp.exp(iv_ref[0]) + 1e-6
    vb = jnp.exp(iv_ref[1]) + 1e-6
    s4 = (jnp.dot(va, w4[:H, :], preferred_element_type=jnp.float32)
          + jnp.dot(vb, w4[H:, :], preferred_element_type=jnp.float32))
    o3[0] = s3[:, :H]
    o3[1] = s3[:, H:]
    o4[0] = s4[:, :H]
    o4[1] = s4[:, H:]


def _mm2(m, ilv, w3, w4):
    grid = (N_NODES // ROW_BLK,)
    pad_spec = pl.BlockSpec((2, ROW_BLK, H), lambda i: (0, i, 0))
    return pl.pallas_call(
        _mm2_body,
        grid=grid,
        in_specs=[pad_spec, pad_spec,
                  pl.BlockSpec((D, D), lambda i: (0, 0)),
                  pl.BlockSpec((D, D), lambda i: (0, 0))],
        out_specs=[_halves_spec()] * 2,
        out_shape=tuple(jax.ShapeDtypeStruct((2, N_NODES, H), jnp.float32)
                        for _ in range(2)),
    )(m, ilv, w3, w4)


def _epilogue_body(nm_ref, nl_ref, mean_ref, std_ref):
    mean_ref[:, :H] = nm_ref[0]
    mean_ref[:, H:] = nm_ref[1]
    std_ref[:, :H] = jnp.sqrt(jnp.exp(nl_ref[0]) + 1e-6)
    std_ref[:, H:] = jnp.sqrt(jnp.exp(nl_ref[1]) + 1e-6)


def _epilogue(nm, nlv):
    grid = (N_NODES // ROW_BLK,)
    pad_spec = pl.BlockSpec((2, ROW_BLK, H), lambda i: (0, i, 0))
    full_spec = pl.BlockSpec((ROW_BLK, D), lambda i: (i, 0))
    return pl.pallas_call(
        _epilogue_body,
        grid=grid,
        in_specs=[pad_spec] * 2,
        out_specs=[full_spec] * 2,
        out_shape=(jax.ShapeDtypeStruct((N_NODES, D), jnp.float32),
                   jax.ShapeDtypeStruct((N_NODES, D), jnp.float32)),
    )(nm, nlv)


def _sc_scatter_body(sup1_hbm, sup2_hbm, src_hbm, dst_hbm, ew_hbm, ew2_hbm,
                     zeros_hbm, out1_hbm, out2_hbm,
                     acc, srcr, dstr, ewr, msg, asem, gsem, ssem):
    c = lax.axis_index("c")
    s = lax.axis_index("s")
    r0 = s * ROWS_PER_TILE
    ebase = s * EDGES_PER_TILE

    def process(sup_hbm, w_hbm):
        def issue_aux(cl, a):
            off = ebase + cl * CHUNK
            pltpu.async_copy(src_hbm.at[pl.ds(off, CHUNK)], srcr.at[a],
                             asem.at[a])
            pltpu.async_copy(dst_hbm.at[pl.ds(off, CHUNK)], dstr.at[a],
                             asem.at[a])
            pltpu.async_copy(w_hbm.at[pl.ds(off, CHUNK)], ewr.at[a],
                             asem.at[a])

        def wait_aux(a):
            pltpu.make_async_copy(src_hbm.at[pl.ds(0, CHUNK)], srcr.at[a],
                                  asem.at[a]).wait()
            pltpu.make_async_copy(dst_hbm.at[pl.ds(0, CHUNK)], dstr.at[a],
                                  asem.at[a]).wait()
            pltpu.make_async_copy(w_hbm.at[pl.ds(0, CHUNK)], ewr.at[a],
                                  asem.at[a]).wait()

        def issue_gather(b, a):
            pltpu.async_copy(sup_hbm.at[srcr.at[a]], msg.at[b], gsem.at[b])

        def wait_gather(b):
            pltpu.make_async_copy(sup_hbm.at[srcr.at[0]], msg.at[b],
                                  gsem.at[b]).wait()

        def issue_scatter(b, a):
            pltpu.async_copy(msg.at[b], acc.at[dstr.at[a]], ssem.at[b],
                             add=True)

        def wait_scatter(b):
            pltpu.make_async_copy(msg.at[b], acc.at[dstr.at[0]],
                                  ssem.at[b]).wait()

        def scale(b, a):
            for g in range(CHUNK // LANES):
                w16 = ewr[a, pl.ds(g * LANES, LANES)]

                def edge16(e16, _):
                    w = w16[jnp.full((LANES,), e16, jnp.int32)]
                    row = g * LANES + e16
                    for j in range(H // LANES):
                        sl = pl.ds(j * LANES, LANES)
                        msg[b, row, sl] = msg[b, row, sl] * w
                    return ()

                lax.fori_loop(0, LANES, edge16, (), unroll=2)

        def step(cl, b, a, static=False):
            b2 = (b + 2) % NBUF
            a2 = (a + 2) % NAUX
            a3 = (a + 3) % NAUX

            def prefetch():
                if static:
                    if cl >= 2:
                        wait_scatter(b2)
                else:
                    @pl.when(cl >= 2)
                    def _():
                        wait_scatter(b2)
                wait_aux(a2)
                issue_gather(b2, a2)

            def prefetch_aux():
                issue_aux(cl + 3, a3)

            if static:
                if cl + 2 < CHUNKS_PER_TILE:
                    prefetch()
                if cl + 3 < CHUNKS_PER_TILE:
                    prefetch_aux()
            else:
                pl.when(cl + 2 < CHUNKS_PER_TILE)(prefetch)
                pl.when(cl + 3 < CHUNKS_PER_TILE)(prefetch_aux)

            wait_gather(b)
            scale(b, a)
            issue_scatter(b, a)

        issue_aux(0, 0)
        issue_aux(1, 1)
        issue_aux(2, 2)
        wait_aux(0)
        issue_gather(0, 0)
        wait_aux(1)
        issue_gather(1, 1)

        def ring_body(i, _):
            for k in range(NAUX):
                cl = i * NAUX + k
                step(cl, k % NBUF, k)
            return ()

        lax.fori_loop(0, CHUNKS_PER_TILE // NAUX, ring_body, ())
        for cl in range((CHUNKS_PER_TILE // NAUX) * NAUX, CHUNKS_PER_TILE):
            step(cl, cl % NBUF, cl % NAUX, static=True)
        for b in range(NBUF):
            wait_scatter(b)

    def do_layer(sup_hbm, w_hbm, out_hbm):
        pltpu.sync_copy(zeros_hbm.at[pl.ds(r0, ROWS_PER_TILE)],
                        acc.at[pl.ds(r0, ROWS_PER_TILE)])
        plsc.subcore_barrier()
        process(sup_hbm.at[c], w_hbm)
        plsc.subcore_barrier()
        pltpu.sync_copy(acc.at[pl.ds(r0, ROWS_PER_TILE)],
                        out_hbm.at[c].at[pl.ds(r0, ROWS_PER_TILE)])

    do_layer(sup1_hbm, ew_hbm, out1_hbm)
    do_layer(sup2_hbm, ew2_hbm, out2_hbm)


_sc_scatter_kernel = functools.partial(
    pl.kernel,
    out_type=(jax.ShapeDtypeStruct((2, N_PAD, H), jnp.float32),
              jax.ShapeDtypeStruct((2, N_PAD, H), jnp.float32)),
    mesh=plsc.VectorSubcoreMesh(core_axis_name="c", subcore_axis_name="s"),
    scratch_types=[
        pltpu.VMEM_SHARED((N_PAD, H), jnp.float32),
        pltpu.VMEM((NAUX, CHUNK), jnp.int32),
        pltpu.VMEM((NAUX, CHUNK), jnp.int32),
        pltpu.VMEM((NAUX, CHUNK), jnp.float32),
        pltpu.VMEM((NBUF, CHUNK, H), jnp.float32),
        pltpu.SemaphoreType.DMA((NAUX,)),
        pltpu.SemaphoreType.DMA((NBUF,)),
        pltpu.SemaphoreType.DMA((NBUF,)),
    ],
)(_sc_scatter_body)


def _sc_scatter2(sup1, sup2, src, dst, ew, ew2, zeros):
    return _sc_scatter_kernel(sup1, sup2, src, dst, ew, ew2, zeros)


def kernel(x, edge_index, edge_weight, mu_im, ls_im, mu_is, ls_is,
           mu_pm, ls_pm, mu_ps, ls_ps):
    src = edge_index[0]
    dst = edge_index[1]
    ek = jax.random.split(jax.random.key(42), 4)
    eps = [jax.random.normal(k, (D, D), dtype=jnp.float32) for k in ek]
    ew = edge_weight
    ew2 = edge_weight * edge_weight
    zeros = jnp.zeros((N_PAD, H), jnp.float32)

    w1, w2, w3, w4, kl = _weights_kl(
        (mu_im, ls_im, eps[0], mu_is, ls_is, eps[1],
         mu_pm, ls_pm, eps[2], mu_ps, ls_ps, eps[3]))

    s1, s2 = _mm1(x, w1, w2)
    m, ilv = _sc_scatter2(s1, s2, src, dst, ew, ew2, zeros)
    s3, s4 = _mm2(m, ilv, w3, w4)
    nm, nlv = _sc_scatter2(s3, s4, src, dst, ew, ew2, zeros)
    new_mean, new_std = _epilogue(nm, nlv)
    return new_mean, new_std, kl[0, 0]


# speedup vs baseline: 1.0084x; 1.0084x over previous
import functools

import jax
import jax.numpy as jnp
from jax import lax
from jax.experimental import pallas as pl
from jax.experimental.pallas import tpu as pltpu
from jax.experimental.pallas import tpu_sc as plsc

N_NODES = 10000
N_EDGES = 160000
D = 256
H = D // 2
NS = 16
CHUNK = 80
EDGES_PER_TILE = N_EDGES // NS
CHUNKS_PER_TILE = EDGES_PER_TILE // CHUNK
NBUF = 4
NAUX = 8
N_PAD = 10240
ROWS_PER_TILE = N_PAD // NS
ROW_BLK = 1000
LANES = 16


def _kl_body(mu1, ls1, mu2, ls2, mu3, ls3, mu4, ls4, kl):
    tot = jnp.float32(0.0)
    for m, l in ((mu1, ls1), (mu2, ls2), (mu3, ls3), (mu4, ls4)):
        lv = l[...]
        mv = m[...]
        tot = tot + jnp.sum(0.5 * (jnp.exp(2.0 * lv) + mv * mv - 2.0 * lv - 1.0))
    kl[...] = jnp.reshape(tot, (1, 1))


def _kl(mus_ls):
    return pl.pallas_call(
        _kl_body,
        out_shape=jax.ShapeDtypeStruct((1, 1), jnp.float32),
    )(*mus_ls)


def _halves_spec():
    return pl.BlockSpec((2, ROW_BLK, H), lambda i: (0, i, 0))


def _mm1_body(x_ref, mu1, ls1, e1, mu2, ls2, e2, o1, o2):
    xv = x_ref[...]
    w1 = mu1[...] + e1[...] * jnp.exp(ls1[...])
    w2 = mu2[...] + e2[...] * jnp.exp(ls2[...])
    s1 = jnp.dot(xv, w1, preferred_element_type=jnp.float32)
    s2 = jnp.dot(xv, w2, preferred_element_type=jnp.float32)
    o1[0] = s1[:, :H]
    o1[1] = s1[:, H:]
    o2[0] = s2[:, :H]
    o2[1] = s2[:, H:]


def _mm1(x, mu1, ls1, e1, mu2, ls2, e2):
    grid = (N_NODES // ROW_BLK,)
    wspec = pl.BlockSpec((D, D), lambda i: (0, 0))
    return pl.pallas_call(
        _mm1_body,
        grid=grid,
        in_specs=[pl.BlockSpec((ROW_BLK, D), lambda i: (i, 0))]
                 + [wspec] * 6,
        out_specs=[_halves_spec()] * 2,
        out_shape=tuple(jax.ShapeDtypeStruct((2, N_NODES, H), jnp.float32)
                        for _ in range(2)),
    )(x, mu1, ls1, e1, mu2, ls2, e2)


def _mm2_body(m_ref, iv_ref, mu3, ls3, e3, mu4, ls4, e4, o3, o4):
    w3 = mu3[...] + e3[...] * jnp.exp(ls3[...])
    w4 = mu4[...] + e4[...] * jnp.exp(ls4[...])
    s3 = (jnp.dot(m_ref[0], w3[:H, :], preferred_element_type=jnp.float32)
          + jnp.dot(m_ref[1], w3[H:, :], preferred_element_type=jnp.float32))
    va = jnp.exp(iv_ref[0]) + 1e-6
    vb = jnp.exp(iv_ref[1]) + 1e-6
    s4 = (jnp.dot(va, w4[:H, :], preferred_element_type=jnp.float32)
          + jnp.dot(vb, w4[H:, :], preferred_element_type=jnp.float32))
    o3[0] = s3[:, :H]
    o3[1] = s3[:, H:]
    o4[0] = s4[:, :H]
    o4[1] = s4[:, H:]


def _mm2(m, ilv, mu3, ls3, e3, mu4, ls4, e4):
    grid = (N_NODES // ROW_BLK,)
    pad_spec = pl.BlockSpec((2, ROW_BLK, H), lambda i: (0, i, 0))
    wspec = pl.BlockSpec((D, D), lambda i: (0, 0))
    return pl.pallas_call(
        _mm2_body,
        grid=grid,
        in_specs=[pad_spec, pad_spec] + [wspec] * 6,
        out_specs=[_halves_spec()] * 2,
        out_shape=tuple(jax.ShapeDtypeStruct((2, N_NODES, H), jnp.float32)
                        for _ in range(2)),
    )(m, ilv, mu3, ls3, e3, mu4, ls4, e4)


def _epilogue_body(nm_ref, nl_ref, mean_ref, std_ref):
    mean_ref[:, :H] = nm_ref[0]
    mean_ref[:, H:] = nm_ref[1]
    std_ref[:, :H] = jnp.sqrt(jnp.exp(nl_ref[0]) + 1e-6)
    std_ref[:, H:] = jnp.sqrt(jnp.exp(nl_ref[1]) + 1e-6)


def _epilogue(nm, nlv):
    grid = (N_NODES // ROW_BLK,)
    pad_spec = pl.BlockSpec((2, ROW_BLK, H), lambda i: (0, i, 0))
    full_spec = pl.BlockSpec((ROW_BLK, D), lambda i: (i, 0))
    return pl.pallas_call(
        _epilogue_body,
        grid=grid,
        in_specs=[pad_spec] * 2,
        out_specs=[full_spec] * 2,
        out_shape=(jax.ShapeDtypeStruct((N_NODES, D), jnp.float32),
                   jax.ShapeDtypeStruct((N_NODES, D), jnp.float32)),
    )(nm, nlv)


def _sc_scatter_body(sup1_hbm, sup2_hbm, src_hbm, dst_hbm, ew_hbm, ew2_hbm,
                     zeros_hbm, out1_hbm, out2_hbm,
                     acc, srcr, dstr, ewr, msg, asem, gsem, ssem):
    c = lax.axis_index("c")
    s = lax.axis_index("s")
    r0 = s * ROWS_PER_TILE
    ebase = s * EDGES_PER_TILE

    def process(sup_hbm, w_hbm):
        def issue_aux(cl, a):
            off = ebase + cl * CHUNK
            pltpu.async_copy(src_hbm.at[pl.ds(off, CHUNK)], srcr.at[a],
                             asem.at[a])
            pltpu.async_copy(dst_hbm.at[pl.ds(off, CHUNK)], dstr.at[a],
                             asem.at[a])
            pltpu.async_copy(w_hbm.at[pl.ds(off, CHUNK)], ewr.at[a],
                             asem.at[a])

        def wait_aux(a):
            pltpu.make_async_copy(src_hbm.at[pl.ds(0, CHUNK)], srcr.at[a],
                                  asem.at[a]).wait()
            pltpu.make_async_copy(dst_hbm.at[pl.ds(0, CHUNK)], dstr.at[a],
                                  asem.at[a]).wait()
            pltpu.make_async_copy(w_hbm.at[pl.ds(0, CHUNK)], ewr.at[a],
                                  asem.at[a]).wait()

        def issue_gather(b, a):
            pltpu.async_copy(sup_hbm.at[srcr.at[a]], msg.at[b], gsem.at[b])

        def wait_gather(b):
            pltpu.make_async_copy(sup_hbm.at[srcr.at[0]], msg.at[b],
                                  gsem.at[b]).wait()

        def issue_scatter(b, a):
            pltpu.async_copy(msg.at[b], acc.at[dstr.at[a]], ssem.at[b],
                             add=True)

        def wait_scatter(b):
            pltpu.make_async_copy(msg.at[b], acc.at[dstr.at[0]],
                                  ssem.at[b]).wait()

        def scale(b, a):
            for g in range(CHUNK // LANES):
                w16 = ewr[a, pl.ds(g * LANES, LANES)]

                def edge16(e16, _):
                    w = w16[jnp.full((LANES,), e16, jnp.int32)]
                    row = g * LANES + e16
                    for j in range(H // LANES):
                        sl = pl.ds(j * LANES, LANES)
                        msg[b, row, sl] = msg[b, row, sl] * w
                    return ()

                lax.fori_loop(0, LANES, edge16, ())

        def step(cl, b, a, static=False):
            b2 = (b + 2) % NBUF
            a2 = (a + 2) % NAUX
            a3 = (a + 3) % NAUX

            def prefetch():
                if static:
                    if cl >= 2:
                        wait_scatter(b2)
                else:
                    @pl.when(cl >= 2)
                    def _():
                        wait_scatter(b2)
                wait_aux(a2)
                issue_gather(b2, a2)

            def prefetch_aux():
                issue_aux(cl + 3, a3)

            if static:
                if cl + 2 < CHUNKS_PER_TILE:
                    prefetch()
                if cl + 3 < CHUNKS_PER_TILE:
                    prefetch_aux()
            else:
                pl.when(cl + 2 < CHUNKS_PER_TILE)(prefetch)
                pl.when(cl + 3 < CHUNKS_PER_TILE)(prefetch_aux)

            wait_gather(b)
            scale(b, a)
            issue_scatter(b, a)

        issue_aux(0, 0)
        issue_aux(1, 1)
        issue_aux(2, 2)
        wait_aux(0)
        issue_gather(0, 0)
        wait_aux(1)
        issue_gather(1, 1)

        def ring_body(i, _):
            for k in range(NAUX):
                cl = i * NAUX + k
                step(cl, k % NBUF, k)
            return ()

        lax.fori_loop(0, CHUNKS_PER_TILE // NAUX, ring_body, ())
        for cl in range((CHUNKS_PER_TILE // NAUX) * NAUX, CHUNKS_PER_TILE):
            step(cl, cl % NBUF, cl % NAUX, static=True)
        for b in range(NBUF):
            wait_scatter(b)

    def do_layer(sup_hbm, w_hbm, out_hbm):
        pltpu.sync_copy(zeros_hbm.at[pl.ds(r0, ROWS_PER_TILE)],
                        acc.at[pl.ds(r0, ROWS_PER_TILE)])
        plsc.subcore_barrier()
        process(sup_hbm.at[c], w_hbm)
        plsc.subcore_barrier()
        pltpu.sync_copy(acc.at[pl.ds(r0, ROWS_PER_TILE)],
                        out_hbm.at[c].at[pl.ds(r0, ROWS_PER_TILE)])

    do_layer(sup1_hbm, ew_hbm, out1_hbm)
    do_layer(sup2_hbm, ew2_hbm, out2_hbm)


_sc_scatter_kernel = functools.partial(
    pl.kernel,
    out_type=(jax.ShapeDtypeStruct((2, N_PAD, H), jnp.float32),
              jax.ShapeDtypeStruct((2, N_PAD, H), jnp.float32)),
    mesh=plsc.VectorSubcoreMesh(core_axis_name="c", subcore_axis_name="s"),
    scratch_types=[
        pltpu.VMEM_SHARED((N_PAD, H), jnp.float32),
        pltpu.VMEM((NAUX, CHUNK), jnp.int32),
        pltpu.VMEM((NAUX, CHUNK), jnp.int32),
        pltpu.VMEM((NAUX, CHUNK), jnp.float32),
        pltpu.VMEM((NBUF, CHUNK, H), jnp.float32),
        pltpu.SemaphoreType.DMA((NAUX,)),
        pltpu.SemaphoreType.DMA((NBUF,)),
        pltpu.SemaphoreType.DMA((NBUF,)),
    ],
)(_sc_scatter_body)


def _sc_scatter2(sup1, sup2, src, dst, ew, ew2, zeros):
    return _sc_scatter_kernel(sup1, sup2, src, dst, ew, ew2, zeros)


def kernel(x, edge_index, edge_weight, mu_im, ls_im, mu_is, ls_is,
           mu_pm, ls_pm, mu_ps, ls_ps):
    src = edge_index[0]
    dst = edge_index[1]
    ek = jax.random.split(jax.random.key(42), 4)
    eps = [jax.random.normal(k, (D, D), dtype=jnp.float32) for k in ek]
    ew = edge_weight
    ew2 = edge_weight * edge_weight
    zeros = jnp.zeros((N_PAD, H), jnp.float32)

    kl = _kl((mu_im, ls_im, mu_is, ls_is, mu_pm, ls_pm, mu_ps, ls_ps))

    s1, s2 = _mm1(x, mu_im, ls_im, eps[0], mu_is, ls_is, eps[1])
    m, ilv = _sc_scatter2(s1, s2, src, dst, ew, ew2, zeros)
    s3, s4 = _mm2(m, ilv, mu_pm, ls_pm, eps[2], mu_ps, ls_ps, eps[3])
    nm, nlv = _sc_scatter2(s3, s4, src, dst, ew, ew2, zeros)
    new_mean, new_std = _epilogue(nm, nlv)
    return new_mean, new_std, kl[0, 0]


# speedup vs baseline: 1.0220x; 1.0135x over previous
import functools

import jax
import jax.numpy as jnp
from jax import lax
from jax.experimental import pallas as pl
from jax.experimental.pallas import tpu as pltpu
from jax.experimental.pallas import tpu_sc as plsc

N_NODES = 10000
N_EDGES = 160000
D = 256
H = D // 2
NS = 16
CHUNK = 80
EDGES_PER_TILE = N_EDGES // NS
CHUNKS_PER_TILE = EDGES_PER_TILE // CHUNK
NBUF = 4
NAUX = 8
N_PAD = 10240
ROWS_PER_TILE = N_PAD // NS
ROW_BLK = 1000
LANES = 16


def _kl_body(mu1, ls1, mu2, ls2, mu3, ls3, mu4, ls4, kl):
    tot = jnp.float32(0.0)
    for m, l in ((mu1, ls1), (mu2, ls2), (mu3, ls3), (mu4, ls4)):
        lv = l[...]
        mv = m[...]
        tot = tot + jnp.sum(0.5 * (jnp.exp(2.0 * lv) + mv * mv - 2.0 * lv - 1.0))
    kl[...] = jnp.reshape(tot, (1, 1))


def _kl(mus_ls):
    return pl.pallas_call(
        _kl_body,
        out_shape=jax.ShapeDtypeStruct((1, 1), jnp.float32),
    )(*mus_ls)


def _halves_spec():
    return pl.BlockSpec((2, ROW_BLK, H), lambda i: (0, i, 0))


def _mm1_body(x_ref, mu1, ls1, e1, mu2, ls2, e2, o1, o2):
    xv = x_ref[...]
    w1 = mu1[...] + e1[...] * jnp.exp(ls1[...])
    w2 = mu2[...] + e2[...] * jnp.exp(ls2[...])
    s1 = jnp.dot(xv, w1, preferred_element_type=jnp.float32)
    s2 = jnp.dot(xv, w2, preferred_element_type=jnp.float32)
    o1[0] = s1[:, :H]
    o1[1] = s1[:, H:]
    o2[0] = s2[:, :H]
    o2[1] = s2[:, H:]


def _mm1(x, mu1, ls1, e1, mu2, ls2, e2):
    grid = (N_NODES // ROW_BLK,)
    wspec = pl.BlockSpec((D, D), lambda i: (0, 0))
    return pl.pallas_call(
        _mm1_body,
        grid=grid,
        in_specs=[pl.BlockSpec((ROW_BLK, D), lambda i: (i, 0))]
                 + [wspec] * 6,
        out_specs=[_halves_spec()] * 2,
        out_shape=tuple(jax.ShapeDtypeStruct((2, N_NODES, H), jnp.float32)
                        for _ in range(2)),
    )(x, mu1, ls1, e1, mu2, ls2, e2)


def _mm2_body(m_ref, iv_ref, mu3, ls3, e3, mu4, ls4, e4, o3, o4):
    w3 = mu3[...] + e3[...] * jnp.exp(ls3[...])
    w4 = mu4[...] + e4[...] * jnp.exp(ls4[...])
    s3 = (jnp.dot(m_ref[0], w3[:H, :], preferred_element_type=jnp.float32)
          + jnp.dot(m_ref[1], w3[H:, :], preferred_element_type=jnp.float32))
    va = jnp.exp(iv_ref[0]) + 1e-6
    vb = jnp.exp(iv_ref[1]) + 1e-6
    s4 = (jnp.dot(va, w4[:H, :], preferred_element_type=jnp.float32)
          + jnp.dot(vb, w4[H:, :], preferred_element_type=jnp.float32))
    o3[0] = s3[:, :H]
    o3[1] = s3[:, H:]
    o4[0] = s4[:, :H]
    o4[1] = s4[:, H:]


def _mm2(m, ilv, mu3, ls3, e3, mu4, ls4, e4):
    grid = (N_NODES // ROW_BLK,)
    pad_spec = pl.BlockSpec((2, ROW_BLK, H), lambda i: (0, i, 0))
    wspec = pl.BlockSpec((D, D), lambda i: (0, 0))
    return pl.pallas_call(
        _mm2_body,
        grid=grid,
        in_specs=[pad_spec, pad_spec] + [wspec] * 6,
        out_specs=[_halves_spec()] * 2,
        out_shape=tuple(jax.ShapeDtypeStruct((2, N_NODES, H), jnp.float32)
                        for _ in range(2)),
    )(m, ilv, mu3, ls3, e3, mu4, ls4, e4)


def _epilogue_body(nm_ref, nl_ref, mean_ref, std_ref):
    mean_ref[:, :H] = nm_ref[0]
    mean_ref[:, H:] = nm_ref[1]
    std_ref[:, :H] = jnp.sqrt(jnp.exp(nl_ref[0]) + 1e-6)
    std_ref[:, H:] = jnp.sqrt(jnp.exp(nl_ref[1]) + 1e-6)


def _epilogue(nm, nlv):
    grid = (N_NODES // ROW_BLK,)
    pad_spec = pl.BlockSpec((2, ROW_BLK, H), lambda i: (0, i, 0))
    full_spec = pl.BlockSpec((ROW_BLK, D), lambda i: (i, 0))
    return pl.pallas_call(
        _epilogue_body,
        grid=grid,
        in_specs=[pad_spec] * 2,
        out_specs=[full_spec] * 2,
        out_shape=(jax.ShapeDtypeStruct((N_NODES, D), jnp.float32),
                   jax.ShapeDtypeStruct((N_NODES, D), jnp.float32)),
    )(nm, nlv)


def _sc_scatter_body(sup1_hbm, sup2_hbm, src_hbm, dst_hbm, ew_hbm, ew2_hbm,
                     zeros_hbm, out1_hbm, out2_hbm,
                     acc, srcr, dstr, ewr, msg, asem, gsem, ssem):
    c = lax.axis_index("c")
    s = lax.axis_index("s")
    r0 = s * ROWS_PER_TILE
    ebase = s * EDGES_PER_TILE

    def process(sup_hbm, w_hbm):
        def issue_aux(cl, a):
            off = ebase + cl * CHUNK
            pltpu.async_copy(src_hbm.at[pl.ds(off, CHUNK)], srcr.at[a],
                             asem.at[a])
            pltpu.async_copy(dst_hbm.at[pl.ds(off, CHUNK)], dstr.at[a],
                             asem.at[a])
            pltpu.async_copy(w_hbm.at[pl.ds(off, CHUNK)], ewr.at[a],
                             asem.at[a])

        def wait_aux(a):
            pltpu.make_async_copy(src_hbm.at[pl.ds(0, CHUNK)], srcr.at[a],
                                  asem.at[a]).wait()
            pltpu.make_async_copy(dst_hbm.at[pl.ds(0, CHUNK)], dstr.at[a],
                                  asem.at[a]).wait()
            pltpu.make_async_copy(w_hbm.at[pl.ds(0, CHUNK)], ewr.at[a],
                                  asem.at[a]).wait()

        def issue_gather(b, a):
            pltpu.async_copy(sup_hbm.at[srcr.at[a]], msg.at[b], gsem.at[b])

        def wait_gather(b):
            pltpu.make_async_copy(sup_hbm.at[srcr.at[0]], msg.at[b],
                                  gsem.at[b]).wait()

        def issue_scatter(b, a):
            pltpu.async_copy(msg.at[b], acc.at[dstr.at[a]], ssem.at[b],
                             add=True)

        def wait_scatter(b):
            pltpu.make_async_copy(msg.at[b], acc.at[dstr.at[0]],
                                  ssem.at[b]).wait()

        def scale(b, a):
            for g in range(CHUNK // LANES):
                w16 = ewr[a, pl.ds(g * LANES, LANES)]

                def edge16(e16, _):
                    w = w16[jnp.full((LANES,), e16, jnp.int32)]
                    row = g * LANES + e16
                    for j in range(H // LANES):
                        sl = pl.ds(j * LANES, LANES)
                        msg[b, row, sl] = msg[b, row, sl] * w
                    return ()

                lax.fori_loop(0, LANES, edge16, ())

        def step(cl, b, a, static=False):
            b2 = (b + 2) % NBUF
            a2 = (a + 2) % NAUX
            a3 = (a + 3) % NAUX

            def prefetch():
                if static:
                    if cl >= 2:
                        wait_scatter(b2)
                else:
                    @pl.when(cl >= 2)
                    def _():
                        wait_scatter(b2)
                wait_aux(a2)
                issue_gather(b2, a2)

            def prefetch_aux():
                issue_aux(cl + 3, a3)

            if static:
                if cl + 2 < CHUNKS_PER_TILE:
                    prefetch()
                if cl + 3 < CHUNKS_PER_TILE:
                    prefetch_aux()
            else:
                pl.when(cl + 2 < CHUNKS_PER_TILE)(prefetch)
                pl.when(cl + 3 < CHUNKS_PER_TILE)(prefetch_aux)

            wait_gather(b)
            scale(b, a)
            issue_scatter(b, a)

        def ring_body(i, _):
            for k in range(NAUX):
                cl = i * NAUX + k
                step(cl, k % NBUF, k)
            return ()

        lax.fori_loop(0, CHUNKS_PER_TILE // NAUX, ring_body, ())
        for cl in range((CHUNKS_PER_TILE // NAUX) * NAUX, CHUNKS_PER_TILE):
            step(cl, cl % NBUF, cl % NAUX, static=True)
        for b in range(NBUF):
            wait_scatter(b)

    def prologue(sup_hbm, w_hbm):
        def _issue_aux(cl, a):
            off = ebase + cl * CHUNK
            pltpu.async_copy(src_hbm.at[pl.ds(off, CHUNK)], srcr.at[a],
                             asem.at[a])
            pltpu.async_copy(dst_hbm.at[pl.ds(off, CHUNK)], dstr.at[a],
                             asem.at[a])
            pltpu.async_copy(w_hbm.at[pl.ds(off, CHUNK)], ewr.at[a],
                             asem.at[a])

        def _wait_aux(a):
            pltpu.make_async_copy(src_hbm.at[pl.ds(0, CHUNK)], srcr.at[a],
                                  asem.at[a]).wait()
            pltpu.make_async_copy(dst_hbm.at[pl.ds(0, CHUNK)], dstr.at[a],
                                  asem.at[a]).wait()
            pltpu.make_async_copy(w_hbm.at[pl.ds(0, CHUNK)], ewr.at[a],
                                  asem.at[a]).wait()

        _issue_aux(0, 0)
        _issue_aux(1, 1)
        _issue_aux(2, 2)
        _wait_aux(0)
        pltpu.async_copy(sup_hbm.at[srcr.at[0]], msg.at[0], gsem.at[0])
        _wait_aux(1)
        pltpu.async_copy(sup_hbm.at[srcr.at[1]], msg.at[1], gsem.at[1])

    sup1 = sup1_hbm.at[c]
    sup2 = sup2_hbm.at[c]
    own = pl.ds(r0, ROWS_PER_TILE)

    prologue(sup1, ew_hbm)
    pltpu.sync_copy(zeros_hbm.at[own], acc.at[own])
    plsc.subcore_barrier()
    process(sup1, ew_hbm)
    prologue(sup2, ew2_hbm)
    plsc.subcore_barrier()
    pltpu.sync_copy(acc.at[own], out1_hbm.at[c].at[own])
    pltpu.sync_copy(zeros_hbm.at[own], acc.at[own])
    plsc.subcore_barrier()
    process(sup2, ew2_hbm)
    plsc.subcore_barrier()
    pltpu.sync_copy(acc.at[own], out2_hbm.at[c].at[own])


_sc_scatter_kernel = functools.partial(
    pl.kernel,
    out_type=(jax.ShapeDtypeStruct((2, N_PAD, H), jnp.float32),
              jax.ShapeDtypeStruct((2, N_PAD, H), jnp.float32)),
    mesh=plsc.VectorSubcoreMesh(core_axis_name="c", subcore_axis_name="s"),
    scratch_types=[
        pltpu.VMEM_SHARED((N_PAD, H), jnp.float32),
        pltpu.VMEM((NAUX, CHUNK), jnp.int32),
        pltpu.VMEM((NAUX, CHUNK), jnp.int32),
        pltpu.VMEM((NAUX, CHUNK), jnp.float32),
        pltpu.VMEM((NBUF, CHUNK, H), jnp.float32),
        pltpu.SemaphoreType.DMA((NAUX,)),
        pltpu.SemaphoreType.DMA((NBUF,)),
        pltpu.SemaphoreType.DMA((NBUF,)),
    ],
)(_sc_scatter_body)


def _sc_scatter2(sup1, sup2, src, dst, ew, ew2, zeros):
    return _sc_scatter_kernel(sup1, sup2, src, dst, ew, ew2, zeros)


def kernel(x, edge_index, edge_weight, mu_im, ls_im, mu_is, ls_is,
           mu_pm, ls_pm, mu_ps, ls_ps):
    src = edge_index[0]
    dst = edge_index[1]
    ek = jax.random.split(jax.random.key(42), 4)
    eps = [jax.random.normal(k, (D, D), dtype=jnp.float32) for k in ek]
    ew = edge_weight
    ew2 = edge_weight * edge_weight
    zeros = jnp.zeros((N_PAD, H), jnp.float32)

    kl = _kl((mu_im, ls_im, mu_is, ls_is, mu_pm, ls_pm, mu_ps, ls_ps))

    s1, s2 = _mm1(x, mu_im, ls_im, eps[0], mu_is, ls_is, eps[1])
    m, ilv = _sc_scatter2(s1, s2, src, dst, ew, ew2, zeros)
    s3, s4 = _mm2(m, ilv, mu_pm, ls_pm, eps[2], mu_ps, ls_ps, eps[3])
    nm, nlv = _sc_scatter2(s3, s4, src, dst, ew, ew2, zeros)
    new_mean, new_std = _epilogue(nm, nlv)
    return new_mean, new_std, kl[0, 0]
